# baseline (speedup 1.0000x reference)
import functools

import jax
import jax.numpy as jnp
from jax import lax
from jax.experimental import pallas as pl
from jax.experimental.pallas import tpu as pltpu
from jax.experimental.pallas import tpu_sc as plsc

D = 64
NC, NS = 2, 16
NW = NC * NS
BB = 128
TB = 8
NBUF = 4
VB = 4096


def _prep_block(wt_ref, out_ref):
    ident = jnp.eye(D, dtype=jnp.float32)
    sw = lax.dot_general(
        wt_ref[...], ident, (((0,), (0,)), ((), ())),
        precision=lax.Precision.HIGHEST,
        preferred_element_type=jnp.float32,
    )
    sw3 = sw.reshape(VB // 2, 2, D)
    out_ref[:, 0:D] = sw3[:, 0, :]
    out_ref[:, D : 2 * D] = sw3[:, 1, :]


def _tc_prep(wt):
    d, v = wt.shape
    return pl.pallas_call(
        _prep_block,
        grid=((v + VB - 1) // VB,),
        in_specs=[pl.BlockSpec((d, VB), lambda i: (0, i))],
        out_specs=pl.BlockSpec((VB // 2, 2 * d), lambda i: (i, 0)),
        out_shape=jax.ShapeDtypeStruct((v // 2, 2 * d), jnp.float32),
    )(wt)


def _make_lookup(t_len: int, b_len: int):
    assert b_len == NW * BB and t_len % TB == 0 and (t_len // TB) % 2 == 1
    ntt = t_len // TB

    mesh = plsc.VectorSubcoreMesh(core_axis_name="c", subcore_axis_name="s")

    @functools.partial(
        pl.kernel,
        out_type=jax.ShapeDtypeStruct((t_len, D // TB, NW, TB, BB), jnp.float32),
        mesh=mesh,
        scratch_types=[
            pltpu.VMEM((TB, BB), jnp.int32),
            pltpu.VMEM((TB, BB), jnp.int32),
            pltpu.VMEM((NBUF, BB, D), jnp.float32),
            pltpu.VMEM((2, D // TB, TB, BB), jnp.float32),
            pltpu.SemaphoreType.DMA,
            pltpu.SemaphoreType.DMA,
            pltpu.SemaphoreType.DMA((NBUF,)),
            pltpu.SemaphoreType.DMA((2,)),
        ],
        compiler_params=pltpu.CompilerParams(
            use_tc_tiling_on_sc=False, needs_layout_passes=False
        ),
    )
    def lookup(xt_hbm, table_hbm, out_hbm, xa, xb, g_v, tr_v,
               sema, semb, gsem, osem):
        w = lax.axis_index("s") * NC + lax.axis_index("c")
        b0 = w * BB
        lanes = lax.iota(jnp.int32, 16)
        kvecs = [
            lax.shift_right_logical(c0 + lanes, 3) for c0 in range(0, D, 16)
        ]
        svecs = [lax.bitwise_and(c0 + lanes, 7) for c0 in range(0, D, 16)]

        def xsrc(tt):
            return xt_hbm.at[pl.ds(tt * TB, TB), pl.ds(b0, BB)]

        def process(tt, xv, xs, pre):
            pltpu.make_async_copy(xsrc(tt), xv, xs).wait()
            if pre is not None:
                ptt, pxv, pxs = pre
                pltpu.async_copy(xsrc(ptt), pxv, pxs)
            gw = [None] * TB
            ow = [None, None]
            for s in range(min(NBUF, TB)):
                gw[s] = pltpu.async_copy(
                    table_hbm.at[xv.at[s]], g_v.at[s % NBUF], gsem.at[s % NBUF]
                )
            for s in range(TB):
                gw[s].wait()
                if ow[s % 2] is not None:
                    ow[s % 2].wait()
                g = g_v.at[s % NBUF]
                tr = tr_v.at[s % 2]

                @plsc.parallel_loop(0, BB, unroll=4)
                def _row(r):
                    rvec = jnp.zeros((16,), jnp.int32) + r
                    for j in range(D // 16):
                        vec = g[r, pl.ds(j * 16, 16)]
                        plsc.store_scatter(
                            tr, [kvecs[j], svecs[j], rvec], vec
                        )

                ow[s % 2] = pltpu.async_copy(
                    tr, out_hbm.at[tt * TB + s, :, w], osem.at[s % 2]
                )
                if s + NBUF < TB:
                    gw[s + NBUF] = pltpu.async_copy(
                        table_hbm.at[xv.at[s + NBUF]],
                        g_v.at[(s + NBUF) % NBUF],
                        gsem.at[(s + NBUF) % NBUF],
                    )
            ow[0].wait()
            ow[1].wait()

        pltpu.async_copy(xsrc(0), xa, sema)

        @pl.loop(0, ntt // 2)
        def _pair(p):
            tt0 = 2 * p
            process(tt0, xa, sema, (tt0 + 1, xb, semb))
            process(tt0 + 1, xb, semb, (tt0 + 2, xa, sema))

        process(ntt - 1, xa, sema, None)

    return lookup


def kernel(x, emb_weight):
    b, t = x.shape
    table2 = _tc_prep(emb_weight.T).reshape(-1, D)
    out5 = _make_lookup(t, b)(x.T.astype(jnp.int32), table2)
    return out5.transpose(2, 4, 0, 1, 3).reshape(b, t, D)


# speedup vs baseline: 1.6333x; 1.6333x over previous
import functools

import jax
import jax.numpy as jnp
from jax import lax
from jax.experimental import pallas as pl
from jax.experimental.pallas import tpu as pltpu
from jax.experimental.pallas import tpu_sc as plsc

D = 64
NC, NS = 2, 16
NW = NC * NS
BB = 128
TB = 8
NBUF = 4
VB = 4096


def _prep_block(wt_ref, out_ref):
    ident = jnp.eye(D, dtype=jnp.float32)
    sw = lax.dot_general(
        wt_ref[...], ident, (((0,), (0,)), ((), ())),
        precision=lax.Precision.HIGHEST,
        preferred_element_type=jnp.float32,
    )
    sw3 = sw.reshape(VB // 2, 2, D)
    out_ref[:, 0:D] = sw3[:, 0, :]
    out_ref[:, D : 2 * D] = sw3[:, 1, :]


def _tc_prep(wt):
    d, v = wt.shape
    return pl.pallas_call(
        _prep_block,
        grid=((v + VB - 1) // VB,),
        in_specs=[pl.BlockSpec((d, VB), lambda i: (0, i))],
        out_specs=pl.BlockSpec((VB // 2, 2 * d), lambda i: (i, 0)),
        out_shape=jax.ShapeDtypeStruct((v // 2, 2 * d), jnp.float32),
    )(wt)


def _make_lookup(t_len: int, b_len: int):
    assert b_len == NW * BB and t_len % TB == 0 and (t_len // TB) % 2 == 1
    ntt = t_len // TB

    mesh = plsc.VectorSubcoreMesh(core_axis_name="c", subcore_axis_name="s")

    @functools.partial(
        pl.kernel,
        out_type=jax.ShapeDtypeStruct((t_len, D // TB, NW, TB, BB), jnp.float32),
        mesh=mesh,
        scratch_types=[
            pltpu.VMEM((TB, BB), jnp.int32),
            pltpu.VMEM((TB, BB), jnp.int32),
            pltpu.VMEM((NBUF, BB, D), jnp.float32),
            pltpu.VMEM((2, D // TB, TB, BB), jnp.float32),
            pltpu.SemaphoreType.DMA,
            pltpu.SemaphoreType.DMA,
            pltpu.SemaphoreType.DMA((NBUF,)),
            pltpu.SemaphoreType.DMA((2,)),
        ],
        compiler_params=pltpu.CompilerParams(
            use_tc_tiling_on_sc=False, needs_layout_passes=False
        ),
    )
    def lookup(xt_hbm, table_hbm, out_hbm, xa, xb, g_v, tr_v,
               sema, semb, gsem, osem):
        w = lax.axis_index("s") * NC + lax.axis_index("c")
        b0 = w * BB
        lanes = lax.iota(jnp.int32, 16)
        kvecs = [
            lax.shift_right_logical(c0 + lanes, 3) for c0 in range(0, D, 16)
        ]
        svecs = [lax.bitwise_and(c0 + lanes, 7) for c0 in range(0, D, 16)]

        def xsrc(tt):
            return xt_hbm.at[pl.ds(tt * TB, TB), pl.ds(b0, BB)]

        def process(tt, xv, xs, pre):
            pltpu.make_async_copy(xsrc(tt), xv, xs).wait()
            if pre is not None:
                ptt, pxv, pxs = pre
                pltpu.async_copy(xsrc(ptt), pxv, pxs)
            gw = [None] * TB
            ow = [None, None]
            for s in range(min(NBUF, TB)):
                gw[s] = pltpu.async_copy(
                    table_hbm.at[xv.at[s]], g_v.at[s % NBUF], gsem.at[s % NBUF]
                )
            for s in range(TB):
                gw[s].wait()
                if ow[s % 2] is not None:
                    ow[s % 2].wait()
                g = g_v.at[s % NBUF]
                tr = tr_v.at[s % 2]

                @plsc.parallel_loop(0, D, unroll=2)
                def _col(c):
                    colv = lax.bitwise_and(c + lanes, D - 1)
                    kv = lax.shift_right_logical(colv, 3)
                    sv = lax.bitwise_and(colv, 7)
                    for i in range(BB // 16):
                        rowv = i * 16 + lanes
                        vec = plsc.load_gather(g, [rowv, colv])
                        plsc.store_scatter(tr, [kv, sv, rowv], vec)

                ow[s % 2] = pltpu.async_copy(
                    tr, out_hbm.at[tt * TB + s, :, w], osem.at[s % 2]
                )
                if s + NBUF < TB:
                    gw[s + NBUF] = pltpu.async_copy(
                        table_hbm.at[xv.at[s + NBUF]],
                        g_v.at[(s + NBUF) % NBUF],
                        gsem.at[(s + NBUF) % NBUF],
                    )
            ow[0].wait()
            ow[1].wait()

        pltpu.async_copy(xsrc(0), xa, sema)

        @pl.loop(0, ntt // 2)
        def _pair(p):
            tt0 = 2 * p
            process(tt0, xa, sema, (tt0 + 1, xb, semb))
            process(tt0 + 1, xb, semb, (tt0 + 2, xa, sema))

        process(ntt - 1, xa, sema, None)

    return lookup


def kernel(x, emb_weight):
    b, t = x.shape
    table2 = _tc_prep(emb_weight.T).reshape(-1, D)
    out5 = _make_lookup(t, b)(x.T.astype(jnp.int32), table2)
    return out5.transpose(2, 4, 0, 1, 3).reshape(b, t, D)
